# Initial kernel scaffold; baseline (speedup 1.0000x reference)
import functools

import jax
import jax.numpy as jnp
from jax import lax
from jax.experimental import pallas as pl
from jax.experimental.pallas import tpu as pltpu
from jax.experimental.pallas import tpu_sc as plsc

_B, _T, _C, _P, _R = 2048, 24, 4, 3, 17
_NC, _NS, _L = 2, 16, 16
_NW = _NC * _NS
_BW = _B // _NW
_NCH = _BW // _L
_NSR = _T * _P * _BW
_IXR = _T * _C * _BW
_MPAD, _BPAD = 160, 56

_LOG1P = (
    -0.017208061121084715,
    0.08172680837495,
    -0.18878267362071732,
    0.31459053537083104,
    -0.49697791116761014,
    0.999792435728606,
    3.5075520536942406e-06,
)


def _softplus(y):
    t = jnp.exp(-jnp.abs(y))
    p = jnp.full((_L,), _LOG1P[0], dtype=jnp.float32)
    for c in _LOG1P[1:]:
        p = p * t + jnp.float32(c)
    return jnp.maximum(y, jnp.float32(0.0)) + p


def _sc_body(ns_hbm, ca_hbm, rl_hbm, m_hbm, be_hbm, out_hbm,
             ns_v, ca_v, rl_v, m_v, be_v):
    wid = lax.axis_index("s") * _NC + lax.axis_index("c")
    pltpu.sync_copy(ns_hbm.at[wid], ns_v)
    pltpu.sync_copy(ca_hbm.at[wid], ca_v)
    pltpu.sync_copy(rl_hbm.at[wid], rl_v)
    pltpu.sync_copy(m_hbm, m_v)
    pltpu.sync_copy(be_hbm, be_v)

    iota = lax.broadcasted_iota(jnp.int32, (_L,), 0)

    def step(i, carry):
        for j in range(_C):
            row = (i * _C + j) * _BW
            for ch in range(_NCH):
                off = row + ch * _L
                ca = ca_v[pl.ds(off, _L)]
                rl = rl_v[pl.ds(off, _L)]
                cbase = ca + (ch * _L + iota)
                c0 = plsc.load_gather(ns_v, [cbase])
                c1 = plsc.load_gather(ns_v, [cbase + _BW])
                c2 = plsc.load_gather(ns_v, [cbase + 2 * _BW])
                m9 = rl * 9
                m = [plsc.load_gather(m_v, [m9 + k]) for k in range(9)]
                msg0 = m[0] * c0 + m[1] * c1 + m[2] * c2
                msg1 = m[3] * c0 + m[4] * c1 + m[5] * c2
                msg2 = m[6] * c0 + m[7] * c1 + m[8] * c2
                pbase = i * (_P * _BW) + ch * _L
                p0 = ns_v[pl.ds(pbase, _L)]
                p1 = ns_v[pl.ds(pbase + _BW, _L)]
                p2 = ns_v[pl.ds(pbase + 2 * _BW, _L)]
                b3 = rl * 3
                a0 = _softplus(p0 * msg0 + plsc.load_gather(be_v, [b3])) + jnp.float32(1e-4)
                a1 = _softplus(p1 * msg1 + plsc.load_gather(be_v, [b3 + 1])) + jnp.float32(1e-4)
                a2 = _softplus(p2 * msg2 + plsc.load_gather(be_v, [b3 + 2])) + jnp.float32(1e-4)
                rinv = jnp.float32(1.0) / (a0 + a1 + a2)
                keep = rl != 0
                ns_v[pl.ds(pbase, _L)] = jnp.where(keep, a0 * rinv, p0)
                ns_v[pl.ds(pbase + _BW, _L)] = jnp.where(keep, a1 * rinv, p1)
                ns_v[pl.ds(pbase + 2 * _BW, _L)] = jnp.where(keep, a2 * rinv, p2)
        return carry

    lax.fori_loop(0, _T, step, 0)
    pltpu.sync_copy(ns_v.at[pl.ds((_T - 1) * _P * _BW, _P * _BW)], out_hbm.at[wid])


@functools.partial(
    pl.kernel,
    out_type=jax.ShapeDtypeStruct((_NW, _P * _BW), jnp.float32),
    mesh=plsc.VectorSubcoreMesh(core_axis_name="c", subcore_axis_name="s",
                                num_cores=_NC, num_subcores=_NS),
    scratch_types=[
        pltpu.VMEM((_NSR,), jnp.float32),
        pltpu.VMEM((_IXR,), jnp.int32),
        pltpu.VMEM((_IXR,), jnp.int32),
        pltpu.VMEM((_MPAD,), jnp.float32),
        pltpu.VMEM((_BPAD,), jnp.float32),
    ],
)
def _sc_kernel(*refs):
    _sc_body(*refs)


def kernel(node_scores, children, rels, labels, M, beta, scale):
    del labels
    ns_w = (node_scores.astype(jnp.float32)
            .transpose(1, 2, 0).reshape(_T * _P, _NW, _BW)
            .transpose(1, 0, 2).reshape(_NW, _NSR))
    ca = (children.astype(jnp.int32) * (_P * _BW))
    ca_w = (ca.transpose(1, 2, 0).reshape(_T * _C, _NW, _BW)
            .transpose(1, 0, 2).reshape(_NW, _IXR))
    rl_w = (rels.astype(jnp.int32)
            .transpose(1, 2, 0).reshape(_T * _C, _NW, _BW)
            .transpose(1, 0, 2).reshape(_NW, _IXR))
    m_flat = jnp.pad((M.astype(jnp.float32) * scale).reshape(_R * _P * _P),
                     (0, _MPAD - _R * _P * _P))
    be_flat = jnp.pad(beta.astype(jnp.float32).reshape(_R * _P),
                      (0, _BPAD - _R * _P))
    out = _sc_kernel(ns_w, ca_w, rl_w, m_flat, be_flat)
    return out.reshape(_NW, _P, _BW).transpose(0, 2, 1).reshape(_B, _P)


# speedup vs baseline: 115.5269x; 115.5269x over previous
import functools

import jax
import jax.numpy as jnp
from jax import lax
from jax.experimental import pallas as pl
from jax.experimental.pallas import tpu as pltpu
from jax.experimental.pallas import tpu_sc as plsc

_B, _T, _C, _P, _R = 2048, 24, 4, 3, 17
_NC, _NS, _L = 2, 16, 16
_NW = _NC * _NS
_BW = _B // _NW
_NCH = _BW // _L
_NSR = _T * _P * _BW
_IXR = _T * _C * _BW
_MPAD, _BPAD = 256, 128
_OPAD = 256
_NSV = (_T - 1) * _P * _BW + _OPAD

_LOG1P = (
    -0.017208061121084715,
    0.08172680837495,
    -0.18878267362071732,
    0.31459053537083104,
    -0.49697791116761014,
    0.999792435728606,
    3.5075520536942406e-06,
)


def _softplus(y):
    t = jnp.exp(-jnp.abs(y))
    p = jnp.full((_L,), _LOG1P[0], dtype=jnp.float32)
    for c in _LOG1P[1:]:
        p = p * t + jnp.float32(c)
    return jnp.maximum(y, jnp.float32(0.0)) + p


def _sc_body(ns_hbm, ca_hbm, rl_hbm, m_hbm, be_hbm, out_hbm,
             ns_v, ca_v, rl_v, m_v, be_v):
    wid = lax.axis_index("s") * _NC + lax.axis_index("c")
    pltpu.sync_copy(ns_hbm.at[wid], ns_v.at[pl.ds(0, _NSR)])
    pltpu.sync_copy(ca_hbm.at[wid], ca_v)
    pltpu.sync_copy(rl_hbm.at[wid], rl_v)
    pltpu.sync_copy(m_hbm, m_v)
    pltpu.sync_copy(be_hbm, be_v)

    iota = lax.broadcasted_iota(jnp.int32, (_L,), 0)

    def step(i, carry):
        for j in range(_C):
            row = (i * _C + j) * _BW
            for ch in range(_NCH):
                off = row + ch * _L
                ca = ca_v[pl.ds(off, _L)]
                rl = rl_v[pl.ds(off, _L)]
                cbase = ca + (ch * _L + iota)
                c0 = plsc.load_gather(ns_v, [cbase])
                c1 = plsc.load_gather(ns_v, [cbase + _BW])
                c2 = plsc.load_gather(ns_v, [cbase + 2 * _BW])
                m9 = rl * 9
                m = [plsc.load_gather(m_v, [m9 + k]) for k in range(9)]
                msg0 = m[0] * c0 + m[1] * c1 + m[2] * c2
                msg1 = m[3] * c0 + m[4] * c1 + m[5] * c2
                msg2 = m[6] * c0 + m[7] * c1 + m[8] * c2
                pbase = i * (_P * _BW) + ch * _L
                p0 = ns_v[pl.ds(pbase, _L)]
                p1 = ns_v[pl.ds(pbase + _BW, _L)]
                p2 = ns_v[pl.ds(pbase + 2 * _BW, _L)]
                b3 = rl * 3
                a0 = _softplus(p0 * msg0 + plsc.load_gather(be_v, [b3])) + jnp.float32(1e-4)
                a1 = _softplus(p1 * msg1 + plsc.load_gather(be_v, [b3 + 1])) + jnp.float32(1e-4)
                a2 = _softplus(p2 * msg2 + plsc.load_gather(be_v, [b3 + 2])) + jnp.float32(1e-4)
                rinv = jnp.float32(1.0) / (a0 + a1 + a2)
                keep = rl != 0
                ns_v[pl.ds(pbase, _L)] = jnp.where(keep, a0 * rinv, p0)
                ns_v[pl.ds(pbase + _BW, _L)] = jnp.where(keep, a1 * rinv, p1)
                ns_v[pl.ds(pbase + 2 * _BW, _L)] = jnp.where(keep, a2 * rinv, p2)
        return carry

    lax.fori_loop(0, _T, step, 0)
    pltpu.sync_copy(ns_v.at[pl.ds((_T - 1) * _P * _BW, _OPAD)], out_hbm.at[wid])


@functools.partial(
    pl.kernel,
    out_type=jax.ShapeDtypeStruct((_NW, _OPAD), jnp.float32),
    mesh=plsc.VectorSubcoreMesh(core_axis_name="c", subcore_axis_name="s",
                                num_cores=_NC, num_subcores=_NS),
    compiler_params=pltpu.CompilerParams(needs_layout_passes=False,
                                         use_tc_tiling_on_sc=False),
    scratch_types=[
        pltpu.VMEM((_NSV,), jnp.float32),
        pltpu.VMEM((_IXR,), jnp.int32),
        pltpu.VMEM((_IXR,), jnp.int32),
        pltpu.VMEM((_MPAD,), jnp.float32),
        pltpu.VMEM((_BPAD,), jnp.float32),
    ],
)
def _sc_kernel(*refs):
    _sc_body(*refs)


def kernel(node_scores, children, rels, labels, M, beta, scale):
    del labels
    ns_w = (node_scores.astype(jnp.float32)
            .transpose(1, 2, 0).reshape(_T * _P, _NW, _BW)
            .transpose(1, 0, 2).reshape(_NW, _NSR))
    ca = (children.astype(jnp.int32) * (_P * _BW))
    ca_w = (ca.transpose(1, 2, 0).reshape(_T * _C, _NW, _BW)
            .transpose(1, 0, 2).reshape(_NW, _IXR))
    rl_w = (rels.astype(jnp.int32)
            .transpose(1, 2, 0).reshape(_T * _C, _NW, _BW)
            .transpose(1, 0, 2).reshape(_NW, _IXR))
    m_flat = jnp.pad((M.astype(jnp.float32) * scale).reshape(_R * _P * _P),
                     (0, _MPAD - _R * _P * _P))
    be_flat = jnp.pad(beta.astype(jnp.float32).reshape(_R * _P),
                      (0, _BPAD - _R * _P))
    out = _sc_kernel(ns_w, ca_w, rl_w, m_flat, be_flat)
    return (out[:, :_P * _BW].reshape(_NW, _P, _BW)
            .transpose(0, 2, 1).reshape(_B, _P))


# speedup vs baseline: 138.9665x; 1.2029x over previous
import functools

import jax
import jax.numpy as jnp
from jax import lax
from jax.experimental import pallas as pl
from jax.experimental.pallas import tpu as pltpu
from jax.experimental.pallas import tpu_sc as plsc

_B, _T, _C, _P, _R = 2048, 24, 4, 3, 17
_NC, _NS, _L = 2, 16, 16
_NW = _NC * _NS
_BW = _B // _NW
_NCH = _BW // _L
_NSR = _T * _P * _BW
_IXR = _T * _C * _BW
_RP = 24
_NTB = _P * _P + _P
_OPAD = 256
_NSV = (_T - 1) * _P * _BW + _OPAD

_C0 = 2.2117031200252768e-05 + 1e-4
_C1 = 0.9990104466294587
_C2 = -0.4891568472023044
_C3 = 0.28330432451740856
_C4 = -0.13011941539126315
_C5 = 0.03010262501167511


def _alpha(y):
    t = jnp.exp(-jnp.abs(y))
    t2 = t * t
    q01 = jnp.float32(_C1) * t + jnp.float32(_C0)
    q23 = jnp.float32(_C3) * t + jnp.float32(_C2)
    q45 = jnp.float32(_C5) * t + jnp.float32(_C4)
    p = (q45 * t2 + q23) * t2 + q01
    return jnp.maximum(y, jnp.float32(0.0)) + p


def _sc_body(ns_hbm, ca_hbm, rl_hbm, tbl_hbm, out_hbm, ns_v, ca_v, rl_v, *tbl):
    wid = lax.axis_index("s") * _NC + lax.axis_index("c")
    pltpu.sync_copy(ns_hbm.at[wid], ns_v.at[pl.ds(0, _NSR)])
    pltpu.sync_copy(ca_hbm.at[wid], ca_v)
    pltpu.sync_copy(rl_hbm.at[wid], rl_v)
    for k in range(_NTB):
        pltpu.sync_copy(tbl_hbm.at[pl.ds(k * _RP, _RP)], tbl[k])

    iota = lax.broadcasted_iota(jnp.int32, (_L,), 0)

    def step(i, carry):
        pb = i * (_P * _BW)
        pmul = i * (_P * _BW)
        par = [[ns_v[pl.ds(pb + q * _BW + ch * _L, _L)] for q in range(_P)]
               for ch in range(_NCH)]
        for j in range(_C):
            row = (i * _C + j) * _BW
            for ch in range(_NCH):
                off = row + ch * _L
                ca = ca_v[pl.ds(off, _L)]
                rl = rl_v[pl.ds(off, _L)]
                p0, p1, p2 = par[ch]
                cbase = ca + (ch * _L + iota)
                self_ref = ca == pmul
                c0 = jnp.where(self_ref, p0, plsc.load_gather(ns_v, [cbase]))
                c1 = jnp.where(self_ref, p1, plsc.load_gather(ns_v, [cbase + _BW]))
                c2 = jnp.where(self_ref, p2, plsc.load_gather(ns_v, [cbase + 2 * _BW]))
                m = [plsc.load_gather(tbl[k], [rl]) for k in range(9)]
                b = [plsc.load_gather(tbl[9 + q], [rl]) for q in range(_P)]
                a0 = _alpha(p0 * (m[0] * c0 + m[1] * c1 + m[2] * c2) + b[0])
                a1 = _alpha(p1 * (m[3] * c0 + m[4] * c1 + m[5] * c2) + b[1])
                a2 = _alpha(p2 * (m[6] * c0 + m[7] * c1 + m[8] * c2) + b[2])
                rinv = jnp.float32(1.0) / (a0 + a1 + a2)
                keep = rl != 0
                par[ch] = [jnp.where(keep, a0 * rinv, p0),
                           jnp.where(keep, a1 * rinv, p1),
                           jnp.where(keep, a2 * rinv, p2)]
        for ch in range(_NCH):
            for q in range(_P):
                ns_v[pl.ds(pb + q * _BW + ch * _L, _L)] = par[ch][q]
        return carry

    lax.fori_loop(0, _T, step, 0)
    pltpu.sync_copy(ns_v.at[pl.ds((_T - 1) * _P * _BW, _OPAD)], out_hbm.at[wid])


@functools.partial(
    pl.kernel,
    out_type=jax.ShapeDtypeStruct((_NW, _OPAD), jnp.float32),
    mesh=plsc.VectorSubcoreMesh(core_axis_name="c", subcore_axis_name="s",
                                num_cores=_NC, num_subcores=_NS),
    compiler_params=pltpu.CompilerParams(needs_layout_passes=False,
                                         use_tc_tiling_on_sc=False),
    scratch_types=[
        pltpu.VMEM((_NSV,), jnp.float32),
        pltpu.VMEM((_IXR,), jnp.int32),
        pltpu.VMEM((_IXR,), jnp.int32),
    ] + [pltpu.VMEM((_RP,), jnp.float32) for _ in range(_NTB)],
)
def _sc_kernel(*refs):
    _sc_body(*refs)


def kernel(node_scores, children, rels, labels, M, beta, scale):
    del labels
    ns_w = (node_scores.astype(jnp.float32)
            .transpose(1, 2, 0).reshape(_T * _P, _NW, _BW)
            .transpose(1, 0, 2).reshape(_NW, _NSR))
    ca = (children.astype(jnp.int32) * (_P * _BW))
    ca_w = (ca.transpose(1, 2, 0).reshape(_T * _C, _NW, _BW)
            .transpose(1, 0, 2).reshape(_NW, _IXR))
    rl_w = (rels.astype(jnp.int32)
            .transpose(1, 2, 0).reshape(_T * _C, _NW, _BW)
            .transpose(1, 0, 2).reshape(_NW, _IXR))
    mt = (M.astype(jnp.float32) * scale).transpose(1, 2, 0).reshape(9, _R)
    bt = beta.astype(jnp.float32).transpose(1, 0)
    tbl = jnp.pad(jnp.concatenate([mt, bt], axis=0), ((0, 0), (0, _RP - _R)))
    out = _sc_kernel(ns_w, ca_w, rl_w, tbl.reshape(_NTB * _RP))
    return (out[:, :_P * _BW].reshape(_NW, _P, _BW)
            .transpose(0, 2, 1).reshape(_B, _P))
